# Initial kernel scaffold; baseline (speedup 1.0000x reference)
import jax
import jax.numpy as jnp
from jax.experimental import pallas as pl


def kernel(species, conv_tensor):
    raise NotImplementedError("write your pallas kernel here")


# speedup vs baseline: 202.0080x; 202.0080x over previous
import functools

import jax
import jax.numpy as jnp
from jax import lax
from jax.experimental import pallas as pl
from jax.experimental.pallas import tpu as pltpu
from jax.experimental.pallas import tpu_sc as plsc

_NC = 2
_NS = 16
_NW = _NC * _NS
_L = 16

_ROWS, _COLS = 16384, 200
_TOTAL = _ROWS * _COLS
_PER_W = _TOTAL // _NW
_CHUNK = 12800
_NCHUNK = _PER_W // _CHUNK
_TBL = 128


@functools.partial(
    pl.kernel,
    mesh=plsc.VectorSubcoreMesh(core_axis_name="c", subcore_axis_name="s"),
    out_type=jax.ShapeDtypeStruct((_TOTAL,), jnp.int32),
    scratch_types=[
        pltpu.VMEM((_TBL,), jnp.int32),
        pltpu.VMEM((_CHUNK,), jnp.int32),
        pltpu.VMEM((_CHUNK,), jnp.int32),
    ],
    compiler_params=pltpu.CompilerParams(needs_layout_passes=False),
)
def _sc_convert(species_hbm, table_hbm, out_hbm, table_v, in_v, out_v):
    wid = lax.axis_index("s") * _NC + lax.axis_index("c")
    base = wid * _PER_W
    pltpu.sync_copy(table_hbm, table_v)

    def chunk_body(c, carry):
        off = base + c * _CHUNK
        pltpu.sync_copy(species_hbm.at[pl.ds(off, _CHUNK)], in_v)

        def body(i, carry2):
            idx = in_v[pl.ds(i * _L, _L)]
            out_v[pl.ds(i * _L, _L)] = plsc.load_gather(table_v, [idx])
            return carry2

        lax.fori_loop(0, _CHUNK // _L, body, 0, unroll=8)
        pltpu.sync_copy(out_v, out_hbm.at[pl.ds(off, _CHUNK)])
        return carry

    lax.fori_loop(0, _NCHUNK, chunk_body, 0)


def kernel(species, conv_tensor):
    table = jnp.pad(conv_tensor, (0, _TBL - conv_tensor.shape[0]))
    out = _sc_convert(species.reshape(-1), table)
    return out.reshape(species.shape)


# speedup vs baseline: 300.4803x; 1.4875x over previous
import functools

import jax
import jax.numpy as jnp
from jax import lax
from jax.experimental import pallas as pl
from jax.experimental.pallas import tpu as pltpu
from jax.experimental.pallas import tpu_sc as plsc

_NC = 2
_NS = 16
_NW = _NC * _NS
_L = 16

_ROWS, _COLS = 16384, 200
_TOTAL = _ROWS * _COLS
_PER_W = _TOTAL // _NW
_CHUNK = 25600
_NCHUNK = _PER_W // _CHUNK
_TBL = 128


@functools.partial(
    pl.kernel,
    mesh=plsc.VectorSubcoreMesh(core_axis_name="c", subcore_axis_name="s"),
    out_type=jax.ShapeDtypeStruct((_TOTAL,), jnp.int32),
    scratch_types=[
        pltpu.VMEM((_TBL,), jnp.int32),
        pltpu.VMEM((_CHUNK,), jnp.int32),
        pltpu.VMEM((_CHUNK,), jnp.int32),
        pltpu.VMEM((_CHUNK,), jnp.int32),
        pltpu.VMEM((_CHUNK,), jnp.int32),
        pltpu.SemaphoreType.DMA,
        pltpu.SemaphoreType.DMA,
        pltpu.SemaphoreType.DMA,
        pltpu.SemaphoreType.DMA,
    ],
    compiler_params=pltpu.CompilerParams(needs_layout_passes=False),
)
def _sc_convert(species_hbm, table_hbm, out_hbm, table_v, in_v0, in_v1,
                out_v0, out_v1, in_sem0, in_sem1, out_sem0, out_sem1):
    in_bufs = (in_v0, in_v1)
    out_bufs = (out_v0, out_v1)
    in_sems = (in_sem0, in_sem1)
    out_sems = (out_sem0, out_sem1)
    wid = lax.axis_index("s") * _NC + lax.axis_index("c")
    base = wid * _PER_W
    pltpu.sync_copy(table_hbm, table_v)

    in_copies = [None] * _NCHUNK
    out_copies = [None] * _NCHUNK
    in_copies[0] = pltpu.async_copy(
        species_hbm.at[pl.ds(base, _CHUNK)], in_bufs[0], in_sems[0])

    for c in range(_NCHUNK):
        slot = c % 2
        in_copies[c].wait()
        if c + 1 < _NCHUNK:
            in_copies[c + 1] = pltpu.async_copy(
                species_hbm.at[pl.ds(base + (c + 1) * _CHUNK, _CHUNK)],
                in_bufs[1 - slot], in_sems[1 - slot])
        if c >= 2:
            out_copies[c - 2].wait()

        in_b = in_bufs[slot]
        out_b = out_bufs[slot]

        @plsc.parallel_loop(0, _CHUNK // _L, 1, unroll=8)
        def _gather(i):
            idx = in_b[pl.ds(i * _L, _L)]
            out_b[pl.ds(i * _L, _L)] = plsc.load_gather(table_v, [idx])

        out_copies[c] = pltpu.async_copy(
            out_b, out_hbm.at[pl.ds(base + c * _CHUNK, _CHUNK)],
            out_sems[slot])

    out_copies[_NCHUNK - 2].wait()
    out_copies[_NCHUNK - 1].wait()


def kernel(species, conv_tensor):
    table = jnp.pad(conv_tensor, (0, _TBL - conv_tensor.shape[0]))
    out = _sc_convert(species.reshape(-1), table)
    return out.reshape(species.shape)


# speedup vs baseline: 510.8651x; 1.7002x over previous
import functools

import jax
import jax.numpy as jnp
from jax import lax
from jax.experimental import pallas as pl
from jax.experimental.pallas import tpu as pltpu
from jax.experimental.pallas import tpu_sc as plsc

_NC = 2
_NS = 16
_NW = _NC * _NS
_L = 16

_ROWS, _COLS = 16384, 200
_ROWS_W = _ROWS // _NW
_CR = 64
_NCHUNK = _ROWS_W // _CR
_TBL = 128

_COL_OFFS = tuple(range(0, _COLS - _L + 1, _L)) + ((_COLS - _L),)


@functools.partial(
    pl.kernel,
    mesh=plsc.VectorSubcoreMesh(core_axis_name="c", subcore_axis_name="s"),
    out_type=jax.ShapeDtypeStruct((_ROWS, _COLS), jnp.int32),
    scratch_types=[
        pltpu.VMEM((_TBL,), jnp.int32),
        pltpu.VMEM((_CR, _COLS), jnp.int32),
        pltpu.VMEM((_CR, _COLS), jnp.int32),
        pltpu.VMEM((_CR, _COLS), jnp.int32),
        pltpu.VMEM((_CR, _COLS), jnp.int32),
        pltpu.SemaphoreType.DMA,
        pltpu.SemaphoreType.DMA,
        pltpu.SemaphoreType.DMA,
        pltpu.SemaphoreType.DMA,
    ],
    compiler_params=pltpu.CompilerParams(needs_layout_passes=False),
)
def _sc_convert(species_hbm, table_hbm, out_hbm, table_v, in_v0, in_v1,
                out_v0, out_v1, in_sem0, in_sem1, out_sem0, out_sem1):
    in_bufs = (in_v0, in_v1)
    out_bufs = (out_v0, out_v1)
    in_sems = (in_sem0, in_sem1)
    out_sems = (out_sem0, out_sem1)
    wid = lax.axis_index("s") * _NC + lax.axis_index("c")
    row0 = wid * _ROWS_W
    pltpu.sync_copy(table_hbm, table_v.at[pl.ds(0, 120)])

    col_vecs = [jnp.full((_L,), c, jnp.int32) + lax.iota(jnp.int32, _L)
                for c in _COL_OFFS]

    in_copies = [None] * _NCHUNK
    out_copies = [None] * _NCHUNK
    in_copies[0] = pltpu.async_copy(
        species_hbm.at[pl.ds(row0, _CR)], in_bufs[0], in_sems[0])

    for ci in range(_NCHUNK):
        slot = ci % 2
        in_copies[ci].wait()
        if ci + 1 < _NCHUNK:
            in_copies[ci + 1] = pltpu.async_copy(
                species_hbm.at[pl.ds(row0 + (ci + 1) * _CR, _CR)],
                in_bufs[1 - slot], in_sems[1 - slot])
        if ci >= 2:
            out_copies[ci - 2].wait()

        in_b = in_bufs[slot]
        out_b = out_bufs[slot]

        @plsc.parallel_loop(0, _CR, 1, unroll=2)
        def _gather(r):
            row_vec = jnp.full((_L,), r, jnp.int32)
            for cv in col_vecs:
                idx = plsc.load_gather(in_b, [row_vec, cv])
                vals = plsc.load_gather(table_v, [idx])
                plsc.store_scatter(out_b, [row_vec, cv], vals)

        out_copies[ci] = pltpu.async_copy(
            out_b, out_hbm.at[pl.ds(row0 + ci * _CR, _CR)], out_sems[slot])

    out_copies[_NCHUNK - 2].wait()
    out_copies[_NCHUNK - 1].wait()


def kernel(species, conv_tensor):
    return _sc_convert(species, conv_tensor)


# speedup vs baseline: 979.9139x; 1.9181x over previous
import functools

import jax
import jax.numpy as jnp
from jax import lax
from jax.experimental import pallas as pl
from jax.experimental.pallas import tpu as pltpu
from jax.experimental.pallas import tpu_sc as plsc

_NC = 2
_NS = 16
_NW = _NC * _NS
_L = 16

_R, _C = 200, 16384
_CW = _C // _NW
_CCH = 128
_NCH = _CW // _CCH
_TBL = 128


@functools.partial(
    pl.kernel,
    mesh=plsc.VectorSubcoreMesh(core_axis_name="c", subcore_axis_name="s"),
    out_type=jax.ShapeDtypeStruct((_R, _C), jnp.int32),
    scratch_types=[
        pltpu.VMEM((_TBL,), jnp.int32),
        pltpu.VMEM((_R, _CCH), jnp.int32),
        pltpu.VMEM((_R, _CCH), jnp.int32),
        pltpu.VMEM((_R, _CCH), jnp.int32),
        pltpu.VMEM((_R, _CCH), jnp.int32),
        pltpu.SemaphoreType.DMA,
        pltpu.SemaphoreType.DMA,
        pltpu.SemaphoreType.DMA,
        pltpu.SemaphoreType.DMA,
    ],
    compiler_params=pltpu.CompilerParams(needs_layout_passes=False),
)
def _sc_convert(st_hbm, table_hbm, out_hbm, table_v, in_v0, in_v1,
                out_v0, out_v1, in_sem0, in_sem1, out_sem0, out_sem1):
    in_bufs = (in_v0, in_v1)
    out_bufs = (out_v0, out_v1)
    in_sems = (in_sem0, in_sem1)
    out_sems = (out_sem0, out_sem1)
    wid = lax.axis_index("s") * _NC + lax.axis_index("c")
    col0 = wid * _CW
    pltpu.sync_copy(table_hbm, table_v.at[pl.ds(0, 120)])

    col_vecs = [jnp.full((_L,), g * _L, jnp.int32) + lax.iota(jnp.int32, _L)
                for g in range(_CCH // _L)]

    in_copies = [None] * _NCH
    out_copies = [None] * _NCH
    in_copies[0] = pltpu.async_copy(
        st_hbm.at[pl.ds(0, _R), pl.ds(col0, _CCH)], in_bufs[0], in_sems[0])

    for ci in range(_NCH):
        slot = ci % 2
        in_copies[ci].wait()
        if ci + 1 < _NCH:
            in_copies[ci + 1] = pltpu.async_copy(
                st_hbm.at[pl.ds(0, _R), pl.ds(col0 + (ci + 1) * _CCH, _CCH)],
                in_bufs[1 - slot], in_sems[1 - slot])
        if ci >= 2:
            out_copies[ci - 2].wait()

        in_b = in_bufs[slot]
        out_b = out_bufs[slot]

        @plsc.parallel_loop(0, _R, 1, unroll=2)
        def _gather(r):
            row_vec = jnp.full((_L,), r, jnp.int32)
            for cv in col_vecs:
                idx = plsc.load_gather(in_b, [row_vec, cv])
                vals = plsc.load_gather(table_v, [idx])
                plsc.store_scatter(out_b, [row_vec, cv], vals)

        out_copies[ci] = pltpu.async_copy(
            out_b, out_hbm.at[pl.ds(0, _R), pl.ds(col0 + ci * _CCH, _CCH)],
            out_sems[slot])

    out_copies[_NCH - 2].wait()
    out_copies[_NCH - 1].wait()


def kernel(species, conv_tensor):
    out_t = _sc_convert(species.T, conv_tensor)
    return out_t.T
